# Initial kernel scaffold; baseline (speedup 1.0000x reference)
import jax
import jax.numpy as jnp
from jax.experimental import pallas as pl


def kernel(x, token_table, pos_table):
    raise NotImplementedError("write your pallas kernel here")


# speedup vs baseline: 1.2473x; 1.2473x over previous
import functools

import jax
import jax.numpy as jnp
from jax import lax
from jax.experimental import pallas as pl
from jax.experimental.pallas import tpu as pltpu
from jax.experimental.pallas import tpu_sc as plsc

_L = 200
_E = 32
_LANES = 16
_W = 128
_NB = 4
_NWORK = 32


def _sc_embed(x2, token_table, pos_table):
    nwin = x2.shape[0]
    wpw = nwin // _NWORK
    B = nwin * _W
    mesh = plsc.VectorSubcoreMesh(core_axis_name="core",
                                  subcore_axis_name="subcore")

    @functools.partial(
        pl.kernel,
        out_type=jax.ShapeDtypeStruct((B, _E), jnp.float32),
        mesh=mesh,
        compiler_params=pltpu.CompilerParams(use_tc_tiling_on_sc=False),
        scratch_types=(
            [pltpu.VMEM((wpw, _W), jnp.int32),
             pltpu.VMEM((2 * _L, _E), jnp.float32),
             pltpu.VMEM((_NB, _W, _E), jnp.float32)]
            + [pltpu.SemaphoreType.DMA] * (2 * _NB)
        ),
    )
    def k(x_hbm, tok_hbm, pos_hbm, out_hbm, idx_all, pos2, bufs, *sems):
        gsem = sems[:_NB]
        wsem = sems[_NB:]
        cid = lax.axis_index("core")
        sid = lax.axis_index("subcore")
        wid = sid * 2 + cid
        base_win = wid * wpw

        pltpu.sync_copy(x_hbm.at[pl.ds(base_win, wpw)], idx_all)
        pltpu.sync_copy(pos_hbm, pos2.at[pl.ds(0, _L)])
        pltpu.sync_copy(pos_hbm, pos2.at[pl.ds(_L, _L)])

        def gather_start(w, b):
            pltpu.async_copy(tok_hbm.at[idx_all.at[w]], bufs.at[b], gsem[b])

        def wb_copy(w, b):
            return pltpu.make_async_copy(
                bufs.at[b], out_hbm.at[pl.ds((base_win + w) * _W, _W)],
                wsem[b])

        gather_start(0, 0)

        @pl.loop(0, wpw, step=_NB)
        def _(g):
            for b in range(_NB):
                w = g + b
                b1 = (b + 1) % _NB

                @pl.when(w + 1 < wpw)
                def _():
                    @pl.when(w + 1 >= _NB)
                    def _():
                        wb_copy(w + 1 - _NB, b1).wait()
                    gather_start(w + 1, b1)

                pltpu.make_async_copy(tok_hbm.at[idx_all.at[w]],
                                      bufs.at[b], gsem[b]).wait()

                phase = lax.rem((base_win + w) * _W, _L)

                @pl.loop(0, _W, step=4)
                def _(r0):
                    for dr in range(4):
                        for h in range(0, _E, _LANES):
                            slc = (pl.ds(r0 + dr, 1), pl.ds(h, _LANES))
                            pslc = (pl.ds(phase + r0 + dr, 1),
                                    pl.ds(h, _LANES))
                            bufs.at[b].at[*slc][...] = (
                                bufs.at[b].at[*slc][...]
                                + pos2.at[*pslc][...])

                wb_copy(w, b).start()

        for b in range(_NB):
            wb_copy(wpw - _NB + b, b).wait()

    return k(x2, token_table, pos_table)


def kernel(x, token_table, pos_table):
    Bseq, L = x.shape
    x2 = x.reshape(-1).astype(jnp.int32).reshape(-1, _W)
    out = _sc_embed(x2, token_table, pos_table)
    return out.reshape(Bseq, L, _E)


# speedup vs baseline: 1.6635x; 1.3337x over previous
import dataclasses
import functools

import jax
import jax.numpy as jnp
from jax import lax
from jax.experimental import pallas as pl
from jax.experimental.pallas import tpu as pltpu
from jax.experimental.pallas import tpu_sc as plsc

_L = 200
_E = 32
_LANES = 16
_W = 128
_NB = 4
_NWORK = 32
_TS = 129


def _compiler_params():
    cp = pltpu.CompilerParams(use_tc_tiling_on_sc=False)
    if "needs_layout_passes" in pltpu.CompilerParams.__dataclass_fields__:
        cp = dataclasses.replace(cp, needs_layout_passes=False)
    return cp


def _sc_embed(x2, token_table, pos_table):
    nwin = x2.shape[0]
    wpw = nwin // _NWORK
    nbatch = x2.shape[1] * 32
    mesh = plsc.VectorSubcoreMesh(core_axis_name="core",
                                  subcore_axis_name="subcore")

    @functools.partial(
        pl.kernel,
        out_type=jax.ShapeDtypeStruct((_L * _E, nbatch), jnp.float32),
        mesh=mesh,
        compiler_params=_compiler_params(),
        scratch_types=(
            [pltpu.VMEM((wpw, _W), jnp.int32),
             pltpu.VMEM((_L, _E), jnp.float32),
             pltpu.VMEM((_NB, _W, _E), jnp.float32),
             pltpu.VMEM((_NB, _E, _TS), jnp.float32)]
            + [pltpu.SemaphoreType.DMA] * (2 * _NB)
        ),
    )
    def k(x_hbm, tok_hbm, pos_hbm, out_hbm, idx_all, pos_v,
          bufs, tbufs, *sems):
        gsem = sems[:_NB]
        wsem = sems[_NB:2 * _NB]
        cid = lax.axis_index("core")
        sid = lax.axis_index("subcore")
        wid = sid * 2 + cid
        base_win = wid * wpw

        pltpu.sync_copy(x_hbm.at[pl.ds(base_win, wpw)], idx_all)
        pltpu.sync_copy(pos_hbm, pos_v)

        iota = lax.iota(jnp.int32, _LANES)
        e_h = [iota + 16 * h for h in range(2)]

        def gather_start(w, b):
            pltpu.async_copy(tok_hbm.at[idx_all.at[w]], bufs.at[b], gsem[b])

        def wb_copy(w_glob, b):
            l = w_glob >> 5
            bt = w_glob & 31
            return pltpu.make_async_copy(
                tbufs.at[b, :, pl.ds(0, _W)],
                out_hbm.at[pl.ds(l * _E, _E), pl.ds(bt * _W, _W)],
                wsem[b])

        gather_start(0, 0)

        @pl.loop(0, wpw, step=_NB)
        def _(g):
            for b in range(_NB):
                w = g + b
                wg = base_win + w

                @pl.when(w >= _NB)
                def _():
                    wb_copy(wg - _NB, b).wait()

                @pl.when(w + 1 < wpw)
                def _():
                    gather_start(w + 1, (b + 1) % _NB)

                pltpu.make_async_copy(tok_hbm.at[idx_all.at[w]],
                                      bufs.at[b], gsem[b]).wait()

                l = wg >> 5
                pv = [pos_v[l, pl.ds(16 * h, _LANES)] for h in range(2)]

                @pl.loop(0, _W, step=4)
                def _(t0):
                    for dt in range(4):
                        col = jnp.full((_LANES,), t0 + dt, jnp.int32)
                        for h in range(2):
                            v = bufs[b, t0 + dt, pl.ds(16 * h, _LANES)]
                            plsc.store_scatter(
                                tbufs.at[b], [e_h[h], col], v + pv[h])

                wb_copy(wg, b).start()

        for b in range(_NB):
            wb_copy(base_win + wpw - _NB + b, b).wait()

    return k(x2, token_table, pos_table)


def kernel(x, token_table, pos_table):
    Bseq, L = x.shape
    x2 = jnp.transpose(x).astype(jnp.int32).reshape(-1, _W)
    p2 = _sc_embed(x2, token_table, pos_table)
    p3 = p2.reshape(L, _E, Bseq)
    return jnp.transpose(p3, (2, 0, 1))
